# baseline (speedup 1.0000x reference)
import functools

import jax
import jax.numpy as jnp
from jax import lax
from jax.experimental import pallas as pl
from jax.experimental.pallas import tpu as pltpu
from jax.experimental.pallas import tpu_sc as plsc

DIM = 256
LANES = 16
NUM_CORES = 2
NUM_SUBCORES = 16
NUM_WORKERS = NUM_CORES * NUM_SUBCORES


@functools.partial(jax.jit, static_argnames=("batch", "seq"))
def _embed(x_flat, token_table, pos_table, batch, seq):
    rows_w = seq // NUM_WORKERS
    mesh = plsc.VectorSubcoreMesh(
        core_axis_name="c", subcore_axis_name="s",
        num_cores=NUM_CORES, num_subcores=NUM_SUBCORES,
    )

    @functools.partial(
        pl.kernel,
        out_type=jax.ShapeDtypeStruct((batch * seq, DIM), jnp.float32),
        mesh=mesh,
        scratch_types=[
            pltpu.VMEM((rows_w,), jnp.int32),
            pltpu.VMEM((rows_w, DIM), jnp.float32),
            pltpu.VMEM((rows_w, DIM), jnp.float32),
            pltpu.SemaphoreType.DMA,
        ],
    )
    def k(x_hbm, tok_hbm, pos_hbm, out_hbm, idx_v, rows_v, pos_v, sem):
        wid = lax.axis_index("s") * NUM_CORES + lax.axis_index("c")
        seq0 = wid * rows_w
        pltpu.sync_copy(pos_hbm.at[pl.ds(seq0, rows_w)], pos_v)

        @pl.loop(0, batch)
        def _batch(b):
            base = b * seq + seq0
            pltpu.sync_copy(x_hbm.at[pl.ds(base, rows_w)], idx_v)
            pltpu.async_copy(tok_hbm.at[idx_v], rows_v, sem).wait()

            @pl.loop(0, rows_w)
            def _row(r):
                @pl.loop(0, DIM, step=LANES)
                def _col(c):
                    sl = (r, pl.ds(c, LANES))
                    rows_v.at[sl][...] = rows_v.at[sl][...] + pos_v.at[sl][...]

            pltpu.sync_copy(rows_v, out_hbm.at[pl.ds(base, rows_w)])

    return k(x_flat, token_table, pos_table)


def kernel(x, token_table, pos_table):
    batch, seq = x.shape
    x_flat = x.reshape(batch * seq).astype(jnp.int32)
    out = _embed(x_flat, token_table, pos_table, batch, seq)
    return out.reshape(batch, seq, DIM)
